# baseline (speedup 1.0000x reference)
import jax
import jax.numpy as jnp
import numpy as np
from jax import lax
from jax.experimental import pallas as pl
from jax.experimental.pallas import tpu as pltpu
from jax.experimental.pallas import tpu_sc as plsc

N = 10000
D = 128
E = 320000
NSPEC = 1000
NSPEC_P = 1024
ALPHA = 1.0

NC, NS = 2, 16
NW = NC * NS
BE = 80
NB = 128
CH = 16
NCH = NB // CH
EP = NW * NB * BE
NPAD = N + 112
RPW = NPAD // NS

BROW = 2000
GRID = N // BROW

_f32 = jnp.float32
_HIGH = lax.Precision.HIGHEST


_MESH = plsc.VectorSubcoreMesh(core_axis_name="c", subcore_axis_name="s")


def _zero_stripe(buf, shared, base):
    @pl.loop(0, BE)
    def _(i):
        for k in range(D // 16):
            buf[i, pl.ds(k * 16, 16)] = jnp.zeros((16,), _f32)

    for k in range(RPW // BE):
        pltpu.sync_copy(buf, shared.at[pl.ds(base + k * BE, BE)])
    rem = RPW % BE
    if rem:
        pltpu.sync_copy(buf.at[pl.ds(0, rem)],
                        shared.at[pl.ds(base + (RPW // BE) * BE, rem)])


def _deg_body(dst_hbm, out_hbm, dst_v, ones_v, hist_sh, sem_s):
    cid = lax.axis_index("c")
    sid = lax.axis_index("s")
    wid = cid * NS + sid
    base = sid * RPW

    _zero_stripe(ones_v, hist_sh, base)

    @pl.loop(0, BE)
    def _(i):
        for k in range(D // 16):
            ones_v[i, pl.ds(k * 16, 16)] = jnp.ones((16,), _f32)

    pltpu.sync_copy(dst_hbm.at[wid], dst_v)
    plsc.subcore_barrier()

    W = 6

    @pl.loop(0, NB)
    def _(b):
        pltpu.async_copy(ones_v, hist_sh.at[dst_v.at[b]], sem_s, add=True)

        @pl.when(b >= W)
        def _():
            pltpu.make_async_copy(ones_v, hist_sh.at[dst_v.at[b]], sem_s).wait()

    for _k in range(W):
        pltpu.make_async_copy(ones_v, hist_sh.at[dst_v.at[0]], sem_s).wait()

    plsc.subcore_barrier()
    pltpu.sync_copy(hist_sh.at[pl.ds(base, RPW)],
                    out_hbm.at[pl.ds(cid * NPAD + base, RPW)])


def _deg_call(dst3):
    f = pl.kernel(
        _deg_body,
        out_type=jax.ShapeDtypeStruct((NC * NPAD, D), _f32),
        mesh=_MESH,
        scratch_types=[
            pltpu.VMEM((NB, BE), jnp.int32),
            pltpu.VMEM((BE, D), _f32),
            pltpu.VMEM_SHARED((NPAD, D), _f32),
            pltpu.SemaphoreType.DMA,
        ],
    )
    return f(dst3)


def _msg_body(hw_hbm, src_hbm, dst_hbm, out_hbm,
              src_a, dst_a, src_b, dst_b, r0, r1, r2, r3, acc_sh,
              g0, g1, g2, g3, s0, s1, s2, s3, semi):
    rows = (r0, r1, r2, r3)
    gs = (g0, g1, g2, g3)
    ss = (s0, s1, s2, s3)
    idxs = ((src_a, dst_a), (src_b, dst_b))
    cid = lax.axis_index("c")
    sid = lax.axis_index("s")
    wid = cid * NS + sid
    base = sid * RPW

    _zero_stripe(r0, acc_sh, base)
    plsc.subcore_barrier()

    def sref(b):
        return idxs[(b // CH) % 2][0].at[b % CH]

    def dref(b):
        return idxs[(b // CH) % 2][1].at[b % CH]

    pltpu.sync_copy(src_hbm.at[wid, pl.ds(0, CH)], src_a)
    pltpu.sync_copy(dst_hbm.at[wid, pl.ds(0, CH)], dst_a)
    pltpu.async_copy(hw_hbm.at[sref(0)], rows[0], gs[0])
    pltpu.async_copy(hw_hbm.at[sref(1)], rows[1], gs[1])

    for i in range(NB):
        j = i % 4
        if i % CH == 0 and i + CH < NB:
            c1 = i // CH + 1
            ns, nd = idxs[c1 % 2]
            pltpu.async_copy(src_hbm.at[wid, pl.ds(c1 * CH, CH)], ns, semi)
            pltpu.async_copy(dst_hbm.at[wid, pl.ds(c1 * CH, CH)], nd, semi)
        if i % CH == CH - 4 and i + CH < NB:
            c1 = i // CH + 1
            ns, nd = idxs[c1 % 2]
            pltpu.make_async_copy(src_hbm.at[wid, pl.ds(c1 * CH, CH)], ns, semi).wait()
            pltpu.make_async_copy(dst_hbm.at[wid, pl.ds(c1 * CH, CH)], nd, semi).wait()

        pltpu.make_async_copy(hw_hbm.at[sref(i)], rows[j], gs[j]).wait()
        pltpu.async_copy(rows[j], acc_sh.at[dref(i)], ss[j], add=True)

        bp = i + 2
        if bp < NB:
            jp = bp % 4
            if bp >= 4:
                pltpu.make_async_copy(rows[jp], acc_sh.at[dref(bp - 4)],
                                      ss[jp]).wait()
            pltpu.async_copy(hw_hbm.at[sref(bp)], rows[jp], gs[jp])

    for b in range(NB - 4, NB):
        j = b % 4
        pltpu.make_async_copy(rows[j], acc_sh.at[dref(b)], ss[j]).wait()

    plsc.subcore_barrier()
    pltpu.sync_copy(acc_sh.at[pl.ds(base, RPW)],
                    out_hbm.at[pl.ds(cid * NPAD + base, RPW)])


def _msg_call(hw, src3, dst3):
    f = pl.kernel(
        _msg_body,
        out_type=jax.ShapeDtypeStruct((NC * NPAD, D), _f32),
        mesh=_MESH,
        scratch_types=[
            pltpu.VMEM((CH, BE), jnp.int32),
            pltpu.VMEM((CH, BE), jnp.int32),
            pltpu.VMEM((CH, BE), jnp.int32),
            pltpu.VMEM((CH, BE), jnp.int32),
            pltpu.VMEM((BE, D), _f32),
            pltpu.VMEM((BE, D), _f32),
            pltpu.VMEM((BE, D), _f32),
            pltpu.VMEM((BE, D), _f32),
            pltpu.VMEM_SHARED((NPAD, D), _f32),
            pltpu.SemaphoreType.DMA,
            pltpu.SemaphoreType.DMA,
            pltpu.SemaphoreType.DMA,
            pltpu.SemaphoreType.DMA,
            pltpu.SemaphoreType.DMA,
            pltpu.SemaphoreType.DMA,
            pltpu.SemaphoreType.DMA,
            pltpu.SemaphoreType.DMA,
            pltpu.SemaphoreType.DMA,
        ],
    )
    return f(hw, src3, dst3)


def _dw_from_coeffs(cpad, ipad):
    r = ipad[:NSPEC_P]
    s = ipad[NSPEC_P:]
    j_a = lax.broadcasted_iota(jnp.int32, (D, NSPEC_P), 0)
    j_b = lax.broadcasted_iota(jnp.int32, (NSPEC_P, D), 1)
    scale = _f32(2.0 * np.pi / D)
    ang_a = ((j_a * r[None, :]) % D).astype(_f32) * scale
    ang_b = ((s[:, None] * j_b) % D).astype(_f32) * scale
    ca = jnp.cos(ang_a) * cpad[None, :]
    sa = jnp.sin(ang_a) * cpad[None, :]
    cb = jnp.cos(ang_b)
    sb = jnp.sin(ang_b)
    dw = (jnp.dot(ca, cb, preferred_element_type=_f32, precision=_HIGH)
          - jnp.dot(sa, sb, preferred_element_type=_f32, precision=_HIGH))
    return dw * _f32(ALPHA / (D * D))


def _fourier_body(x_ref, w1_ref, c1_ref, i1_ref, c2_ref, i2_ref,
                  hw1_ref, embf_ref, dw1_s, dw2_s):
    @pl.when(pl.program_id(0) == 0)
    def _():
        dw1_s[...] = _dw_from_coeffs(c1_ref[0], i1_ref[0])
        dw2_s[...] = _dw_from_coeffs(c2_ref[0], i2_ref[0])

    xb = x_ref[...]
    hw1_ref[...] = jnp.dot(xb, w1_ref[...],
                           preferred_element_type=_f32, precision=_HIGH)
    xf = jnp.maximum(
        jnp.dot(xb, dw1_s[...], preferred_element_type=_f32, precision=_HIGH),
        0.0)
    embf_ref[...] = jnp.dot(xf, dw2_s[...],
                            preferred_element_type=_f32, precision=_HIGH)


def _fourier_call(x, w1, cp1, ip1, cp2, ip2):
    full = lambda shape: pl.BlockSpec(shape, lambda i: (0, 0))
    return pl.pallas_call(
        _fourier_body,
        grid=(GRID,),
        in_specs=[
            pl.BlockSpec((BROW, D), lambda i: (i, 0)),
            full((D, D)),
            full((1, NSPEC_P)),
            full((1, 2 * NSPEC_P)),
            full((1, NSPEC_P)),
            full((1, 2 * NSPEC_P)),
        ],
        out_specs=[
            pl.BlockSpec((BROW, D), lambda i: (i, 0)),
            pl.BlockSpec((BROW, D), lambda i: (i, 0)),
        ],
        out_shape=[
            jax.ShapeDtypeStruct((N, D), _f32),
            jax.ShapeDtypeStruct((N, D), _f32),
        ],
        scratch_shapes=[
            pltpu.VMEM((D, D), _f32),
            pltpu.VMEM((D, D), _f32),
        ],
    )(x, w1, cp1, ip1, cp2, ip2)


def _prep_body(h0_ref, h1_ref, hw1_ref, dinv_ref, hw1s_ref):
    deg = h0_ref[:, 0:1] + h1_ref[:, 0:1] + 1.0
    dinv = lax.rsqrt(deg)
    dinv_b = jnp.broadcast_to(dinv, hw1_ref.shape)
    dinv_ref[...] = dinv_b
    hw1s_ref[...] = hw1_ref[...] * dinv_b


def _prep_call(h0, h1, hw1):
    return pl.pallas_call(
        _prep_body,
        grid=(GRID,),
        in_specs=[
            pl.BlockSpec((BROW, D), lambda i: (i, 0)),
            pl.BlockSpec((BROW, D), lambda i: (i, 0)),
            pl.BlockSpec((BROW, D), lambda i: (i, 0)),
        ],
        out_specs=[
            pl.BlockSpec((BROW, D), lambda i: (i, 0)),
            pl.BlockSpec((BROW, D), lambda i: (i, 0)),
        ],
        out_shape=[
            jax.ShapeDtypeStruct((N, D), _f32),
            jax.ShapeDtypeStruct((N, D), _f32),
        ],
    )(h0, h1, hw1)


def _mid_body(a0_ref, a1_ref, hw1s_ref, dinv_ref, b1_ref, w2_ref, hw2s_ref):
    h1 = jnp.maximum(
        dinv_ref[...] * (a0_ref[...] + a1_ref[...] + hw1s_ref[...])
        + b1_ref[...], 0.0)
    hw2s_ref[...] = dinv_ref[...] * jnp.dot(
        h1, w2_ref[...], preferred_element_type=_f32, precision=_HIGH)


def _mid_call(a0, a1, hw1s, dinv, b1r, w2):
    row = pl.BlockSpec((BROW, D), lambda i: (i, 0))
    return pl.pallas_call(
        _mid_body,
        grid=(GRID,),
        in_specs=[row, row, row, row,
                  pl.BlockSpec((1, D), lambda i: (0, 0)),
                  pl.BlockSpec((D, D), lambda i: (0, 0))],
        out_specs=row,
        out_shape=jax.ShapeDtypeStruct((N, D), _f32),
    )(a0, a1, hw1s, dinv, b1r, w2)


def _final_body(a0_ref, a1_ref, hw2s_ref, dinv_ref, b2_ref, embf_ref,
                base_ref, total_ref):
    base = (dinv_ref[...] * (a0_ref[...] + a1_ref[...] + hw2s_ref[...])
            + b2_ref[...])
    base_ref[...] = base
    total_ref[...] = base + embf_ref[...]


def _final_call(a0, a1, hw2s, dinv, b2r, embf):
    row = pl.BlockSpec((BROW, D), lambda i: (i, 0))
    return pl.pallas_call(
        _final_body,
        grid=(GRID,),
        in_specs=[row, row, row, row,
                  pl.BlockSpec((1, D), lambda i: (0, 0)),
                  row],
        out_specs=[row, row],
        out_shape=[
            jax.ShapeDtypeStruct((N, D), _f32),
            jax.ShapeDtypeStruct((N, D), _f32),
        ],
    )(a0, a1, hw2s, dinv, b2r, embf)


def kernel(x, edge_index, W1, b1, W2, b2, c1, c2, idx1, idx2):
    src = edge_index[0]
    dst = edge_index[1]
    npad = EP - E
    pad_ids = jnp.arange(npad, dtype=jnp.int32)
    psrc = jnp.concatenate([src, pad_ids % 997])
    pdst = jnp.concatenate([dst, N + (pad_ids % 112)])
    src3 = psrc.reshape(NW, NB, BE)
    dst3 = pdst.reshape(NW, NB, BE)

    def pack(c, idx):
        cp = jnp.zeros((1, NSPEC_P), _f32).at[0, :NSPEC].set(c)
        ip = jnp.zeros((1, 2 * NSPEC_P), jnp.int32)
        ip = ip.at[0, :NSPEC].set(idx[0]).at[0, NSPEC_P:NSPEC_P + NSPEC].set(idx[1])
        return cp, ip

    cp1, ip1 = pack(c1, idx1)
    cp2, ip2 = pack(c2, idx2)

    hist = _deg_call(dst3)
    hw1, embf = _fourier_call(x, W1, cp1, ip1, cp2, ip2)

    h0 = hist[:N]
    h1 = hist[NPAD:NPAD + N]
    dinv, hw1s = _prep_call(h0, h1, hw1)

    def msg(hw):
        acc = _msg_call(hw, src3, dst3)
        return acc[:N], acc[NPAD:NPAD + N]

    a10, a11 = msg(hw1s)
    hw2s = _mid_call(a10, a11, hw1s, dinv, b1.reshape(1, D), W2)

    a20, a21 = msg(hw2s)
    emb_base, emb_total = _final_call(a20, a21, hw2s, dinv,
                                      b2.reshape(1, D), embf)
    return (emb_total, emb_base, embf)


# speedup vs baseline: 1.0576x; 1.0576x over previous
import jax
import jax.numpy as jnp
import numpy as np
from jax import lax
from jax.experimental import pallas as pl
from jax.experimental.pallas import tpu as pltpu
from jax.experimental.pallas import tpu_sc as plsc

N = 10000
D = 128
E = 320000
NSPEC = 1000
NSPEC_P = 1024
ALPHA = 1.0

NC, NS = 2, 16
NW = NC * NS
BE = 80
NB = 128
CH = 16
NCH = NB // CH
EP = NW * NB * BE
NPAD = N + 112
RPW = NPAD // NS

BROW = 2000
GRID = N // BROW

_f32 = jnp.float32
_HIGH = lax.Precision.HIGHEST


_MESH = plsc.VectorSubcoreMesh(core_axis_name="c", subcore_axis_name="s")


def _zero_stripe(buf, shared, base):
    @pl.loop(0, BE)
    def _(i):
        for k in range(D // 16):
            buf[i, pl.ds(k * 16, 16)] = jnp.zeros((16,), _f32)

    for k in range(RPW // BE):
        pltpu.sync_copy(buf, shared.at[pl.ds(base + k * BE, BE)])
    rem = RPW % BE
    if rem:
        pltpu.sync_copy(buf.at[pl.ds(0, rem)],
                        shared.at[pl.ds(base + (RPW // BE) * BE, rem)])


def _deg_body(dst_hbm, out0_hbm, out1_hbm, dst_v, ones_v, hist_sh, sem_s):
    cid = lax.axis_index("c")
    sid = lax.axis_index("s")
    wid = cid * NS + sid
    base = sid * RPW

    _zero_stripe(ones_v, hist_sh, base)

    @pl.loop(0, BE)
    def _(i):
        for k in range(D // 16):
            ones_v[i, pl.ds(k * 16, 16)] = jnp.ones((16,), _f32)

    pltpu.sync_copy(dst_hbm.at[wid], dst_v)
    plsc.subcore_barrier()

    W = 6

    @pl.loop(0, NB)
    def _(b):
        pltpu.async_copy(ones_v, hist_sh.at[dst_v.at[b]], sem_s, add=True)

        @pl.when(b >= W)
        def _():
            pltpu.make_async_copy(ones_v, hist_sh.at[dst_v.at[b]], sem_s).wait()

    for _k in range(W):
        pltpu.make_async_copy(ones_v, hist_sh.at[dst_v.at[0]], sem_s).wait()

    plsc.subcore_barrier()

    @pl.when(cid == 0)
    def _():
        pltpu.sync_copy(hist_sh.at[pl.ds(base, RPW)],
                        out0_hbm.at[pl.ds(base, RPW)])

    @pl.when(cid == 1)
    def _():
        pltpu.sync_copy(hist_sh.at[pl.ds(base, RPW)],
                        out1_hbm.at[pl.ds(base, RPW)])


def _deg_call(dst3):
    f = pl.kernel(
        _deg_body,
        out_type=[jax.ShapeDtypeStruct((NPAD, D), _f32),
                  jax.ShapeDtypeStruct((NPAD, D), _f32)],
        mesh=_MESH,
        scratch_types=[
            pltpu.VMEM((NB, BE), jnp.int32),
            pltpu.VMEM((BE, D), _f32),
            pltpu.VMEM_SHARED((NPAD, D), _f32),
            pltpu.SemaphoreType.DMA,
        ],
    )
    return f(dst3)


def _msg_body(hw_hbm, src_hbm, dst_hbm, out0_hbm, out1_hbm,
              src_a, dst_a, src_b, dst_b, r0, r1, r2, r3, acc_sh,
              g0, g1, g2, g3, s0, s1, s2, s3, semi):
    rows = (r0, r1, r2, r3)
    gs = (g0, g1, g2, g3)
    ss = (s0, s1, s2, s3)
    idxs = ((src_a, dst_a), (src_b, dst_b))
    cid = lax.axis_index("c")
    sid = lax.axis_index("s")
    wid = cid * NS + sid
    base = sid * RPW

    _zero_stripe(r0, acc_sh, base)
    plsc.subcore_barrier()

    def sref(b):
        return idxs[(b // CH) % 2][0].at[b % CH]

    def dref(b):
        return idxs[(b // CH) % 2][1].at[b % CH]

    pltpu.sync_copy(src_hbm.at[wid, pl.ds(0, CH)], src_a)
    pltpu.sync_copy(dst_hbm.at[wid, pl.ds(0, CH)], dst_a)
    pltpu.async_copy(hw_hbm.at[sref(0)], rows[0], gs[0])
    pltpu.async_copy(hw_hbm.at[sref(1)], rows[1], gs[1])

    for i in range(NB):
        j = i % 4
        if i % CH == 0 and i + CH < NB:
            c1 = i // CH + 1
            ns, nd = idxs[c1 % 2]
            pltpu.async_copy(src_hbm.at[wid, pl.ds(c1 * CH, CH)], ns, semi)
            pltpu.async_copy(dst_hbm.at[wid, pl.ds(c1 * CH, CH)], nd, semi)
        if i % CH == CH - 4 and i + CH < NB:
            c1 = i // CH + 1
            ns, nd = idxs[c1 % 2]
            pltpu.make_async_copy(src_hbm.at[wid, pl.ds(c1 * CH, CH)], ns, semi).wait()
            pltpu.make_async_copy(dst_hbm.at[wid, pl.ds(c1 * CH, CH)], nd, semi).wait()

        pltpu.make_async_copy(hw_hbm.at[sref(i)], rows[j], gs[j]).wait()
        pltpu.async_copy(rows[j], acc_sh.at[dref(i)], ss[j], add=True)

        bp = i + 2
        if bp < NB:
            jp = bp % 4
            if bp >= 4:
                pltpu.make_async_copy(rows[jp], acc_sh.at[dref(bp - 4)],
                                      ss[jp]).wait()
            pltpu.async_copy(hw_hbm.at[sref(bp)], rows[jp], gs[jp])

    for b in range(NB - 4, NB):
        j = b % 4
        pltpu.make_async_copy(rows[j], acc_sh.at[dref(b)], ss[j]).wait()

    plsc.subcore_barrier()

    @pl.when(cid == 0)
    def _():
        pltpu.sync_copy(acc_sh.at[pl.ds(base, RPW)],
                        out0_hbm.at[pl.ds(base, RPW)])

    @pl.when(cid == 1)
    def _():
        pltpu.sync_copy(acc_sh.at[pl.ds(base, RPW)],
                        out1_hbm.at[pl.ds(base, RPW)])


def _msg_call(hw, src3, dst3):
    f = pl.kernel(
        _msg_body,
        out_type=[jax.ShapeDtypeStruct((NPAD, D), _f32),
                  jax.ShapeDtypeStruct((NPAD, D), _f32)],
        mesh=_MESH,
        scratch_types=[
            pltpu.VMEM((CH, BE), jnp.int32),
            pltpu.VMEM((CH, BE), jnp.int32),
            pltpu.VMEM((CH, BE), jnp.int32),
            pltpu.VMEM((CH, BE), jnp.int32),
            pltpu.VMEM((BE, D), _f32),
            pltpu.VMEM((BE, D), _f32),
            pltpu.VMEM((BE, D), _f32),
            pltpu.VMEM((BE, D), _f32),
            pltpu.VMEM_SHARED((NPAD, D), _f32),
            pltpu.SemaphoreType.DMA,
            pltpu.SemaphoreType.DMA,
            pltpu.SemaphoreType.DMA,
            pltpu.SemaphoreType.DMA,
            pltpu.SemaphoreType.DMA,
            pltpu.SemaphoreType.DMA,
            pltpu.SemaphoreType.DMA,
            pltpu.SemaphoreType.DMA,
            pltpu.SemaphoreType.DMA,
        ],
    )
    return f(hw, src3, dst3)


def _dw_from_coeffs(cpad, ipad):
    r = ipad[:NSPEC_P]
    s = ipad[NSPEC_P:]
    j_a = lax.broadcasted_iota(jnp.int32, (D, NSPEC_P), 0)
    j_b = lax.broadcasted_iota(jnp.int32, (NSPEC_P, D), 1)
    scale = _f32(2.0 * np.pi / D)
    ang_a = ((j_a * r[None, :]) % D).astype(_f32) * scale
    ang_b = ((s[:, None] * j_b) % D).astype(_f32) * scale
    ca = jnp.cos(ang_a) * cpad[None, :]
    sa = jnp.sin(ang_a) * cpad[None, :]
    cb = jnp.cos(ang_b)
    sb = jnp.sin(ang_b)
    dw = (jnp.dot(ca, cb, preferred_element_type=_f32, precision=_HIGH)
          - jnp.dot(sa, sb, preferred_element_type=_f32, precision=_HIGH))
    return dw * _f32(ALPHA / (D * D))


def _fourier_body(x_ref, w1_ref, c1_ref, i1_ref, c2_ref, i2_ref,
                  hw1_ref, embf_ref, dw1_s, dw2_s):
    @pl.when(pl.program_id(0) == 0)
    def _():
        dw1_s[...] = _dw_from_coeffs(c1_ref[0], i1_ref[0])
        dw2_s[...] = _dw_from_coeffs(c2_ref[0], i2_ref[0])

    xb = x_ref[...]
    hw1_ref[...] = jnp.dot(xb, w1_ref[...],
                           preferred_element_type=_f32, precision=_HIGH)
    xf = jnp.maximum(
        jnp.dot(xb, dw1_s[...], preferred_element_type=_f32, precision=_HIGH),
        0.0)
    embf_ref[...] = jnp.dot(xf, dw2_s[...],
                            preferred_element_type=_f32, precision=_HIGH)


def _fourier_call(x, w1, cp1, ip1, cp2, ip2):
    full = lambda shape: pl.BlockSpec(shape, lambda i: (0, 0))
    return pl.pallas_call(
        _fourier_body,
        grid=(GRID,),
        in_specs=[
            pl.BlockSpec((BROW, D), lambda i: (i, 0)),
            full((D, D)),
            full((1, NSPEC_P)),
            full((1, 2 * NSPEC_P)),
            full((1, NSPEC_P)),
            full((1, 2 * NSPEC_P)),
        ],
        out_specs=[
            pl.BlockSpec((BROW, D), lambda i: (i, 0)),
            pl.BlockSpec((BROW, D), lambda i: (i, 0)),
        ],
        out_shape=[
            jax.ShapeDtypeStruct((N, D), _f32),
            jax.ShapeDtypeStruct((N, D), _f32),
        ],
        scratch_shapes=[
            pltpu.VMEM((D, D), _f32),
            pltpu.VMEM((D, D), _f32),
        ],
    )(x, w1, cp1, ip1, cp2, ip2)


def _prep_body(h0_ref, h1_ref, hw1_ref, dinv_ref, hw1s_ref):
    deg = h0_ref[:, 0:1] + h1_ref[:, 0:1] + 1.0
    dinv = lax.rsqrt(deg)
    dinv_b = jnp.broadcast_to(dinv, hw1_ref.shape)
    dinv_ref[...] = dinv_b
    hw1s_ref[...] = hw1_ref[...] * dinv_b


def _prep_call(h0, h1, hw1):
    return pl.pallas_call(
        _prep_body,
        grid=(GRID,),
        in_specs=[
            pl.BlockSpec((BROW, D), lambda i: (i, 0)),
            pl.BlockSpec((BROW, D), lambda i: (i, 0)),
            pl.BlockSpec((BROW, D), lambda i: (i, 0)),
        ],
        out_specs=[
            pl.BlockSpec((BROW, D), lambda i: (i, 0)),
            pl.BlockSpec((BROW, D), lambda i: (i, 0)),
        ],
        out_shape=[
            jax.ShapeDtypeStruct((N, D), _f32),
            jax.ShapeDtypeStruct((N, D), _f32),
        ],
    )(h0, h1, hw1)


def _mid_body(a0_ref, a1_ref, hw1s_ref, dinv_ref, b1_ref, w2_ref, hw2s_ref):
    h1 = jnp.maximum(
        dinv_ref[...] * (a0_ref[...] + a1_ref[...] + hw1s_ref[...])
        + b1_ref[...], 0.0)
    hw2s_ref[...] = dinv_ref[...] * jnp.dot(
        h1, w2_ref[...], preferred_element_type=_f32, precision=_HIGH)


def _mid_call(a0, a1, hw1s, dinv, b1r, w2):
    row = pl.BlockSpec((BROW, D), lambda i: (i, 0))
    return pl.pallas_call(
        _mid_body,
        grid=(GRID,),
        in_specs=[row, row, row, row,
                  pl.BlockSpec((1, D), lambda i: (0, 0)),
                  pl.BlockSpec((D, D), lambda i: (0, 0))],
        out_specs=row,
        out_shape=jax.ShapeDtypeStruct((N, D), _f32),
    )(a0, a1, hw1s, dinv, b1r, w2)


def _final_body(a0_ref, a1_ref, hw2s_ref, dinv_ref, b2_ref, embf_ref,
                base_ref, total_ref):
    base = (dinv_ref[...] * (a0_ref[...] + a1_ref[...] + hw2s_ref[...])
            + b2_ref[...])
    base_ref[...] = base
    total_ref[...] = base + embf_ref[...]


def _final_call(a0, a1, hw2s, dinv, b2r, embf):
    row = pl.BlockSpec((BROW, D), lambda i: (i, 0))
    return pl.pallas_call(
        _final_body,
        grid=(GRID,),
        in_specs=[row, row, row, row,
                  pl.BlockSpec((1, D), lambda i: (0, 0)),
                  row],
        out_specs=[row, row],
        out_shape=[
            jax.ShapeDtypeStruct((N, D), _f32),
            jax.ShapeDtypeStruct((N, D), _f32),
        ],
    )(a0, a1, hw2s, dinv, b2r, embf)


def kernel(x, edge_index, W1, b1, W2, b2, c1, c2, idx1, idx2):
    src = edge_index[0]
    dst = edge_index[1]
    npad = EP - E
    pad_ids = jnp.arange(npad, dtype=jnp.int32)
    psrc = jnp.concatenate([src, pad_ids % 997])
    pdst = jnp.concatenate([dst, N + (pad_ids % 112)])
    src3 = psrc.reshape(NW, NB, BE)
    dst3 = pdst.reshape(NW, NB, BE)

    def pack(c, idx):
        cp = jnp.zeros((1, NSPEC_P), _f32).at[0, :NSPEC].set(c)
        ip = jnp.zeros((1, 2 * NSPEC_P), jnp.int32)
        ip = ip.at[0, :NSPEC].set(idx[0]).at[0, NSPEC_P:NSPEC_P + NSPEC].set(idx[1])
        return cp, ip

    cp1, ip1 = pack(c1, idx1)
    cp2, ip2 = pack(c2, idx2)

    h0, h1 = _deg_call(dst3)
    hw1, embf = _fourier_call(x, W1, cp1, ip1, cp2, ip2)

    dinv, hw1s = _prep_call(h0, h1, hw1)

    a10, a11 = _msg_call(hw1s, src3, dst3)
    hw2s = _mid_call(a10, a11, hw1s, dinv, b1.reshape(1, D), W2)

    a20, a21 = _msg_call(hw2s, src3, dst3)
    emb_base, emb_total = _final_call(a20, a21, hw2s, dinv,
                                      b2.reshape(1, D), embf)
    return (emb_total, emb_base, embf)


# speedup vs baseline: 1.1139x; 1.0532x over previous
import jax
import jax.numpy as jnp
import numpy as np
from jax import lax
from jax.experimental import pallas as pl
from jax.experimental.pallas import tpu as pltpu
from jax.experimental.pallas import tpu_sc as plsc

N = 10000
D = 128
E = 320000
NSPEC = 1000
NSPEC_P = 1024
ALPHA = 1.0

NC, NS = 2, 16
NW = NC * NS
BE = 80
NB = 128
CH = 16
NCH = NB // CH
EP = NW * NB * BE
NPAD = N + 112
RPW = NPAD // NS

BROW = 2000
GRID = N // BROW

_f32 = jnp.float32
_HIGH = lax.Precision.HIGHEST


_MESH = plsc.VectorSubcoreMesh(core_axis_name="c", subcore_axis_name="s")


def _zero_stripe(buf, shared, base):
    @pl.loop(0, BE)
    def _(i):
        for k in range(D // 16):
            buf[i, pl.ds(k * 16, 16)] = jnp.zeros((16,), _f32)

    for k in range(RPW // BE):
        pltpu.sync_copy(buf, shared.at[pl.ds(base + k * BE, BE)])
    rem = RPW % BE
    if rem:
        pltpu.sync_copy(buf.at[pl.ds(0, rem)],
                        shared.at[pl.ds(base + (RPW // BE) * BE, rem)])


def _deg_body(dst_hbm, out_hbm, dst_v, hist_v, red_v):
    cid = lax.axis_index("c")
    sid = lax.axis_index("s")
    wid = cid * NS + sid

    @pl.loop(0, 8 * NPAD // 16)
    def _(g):
        hist_v[pl.ds(g * 16, 16)] = jnp.zeros((16,), _f32)

    pltpu.sync_copy(dst_hbm.at[wid], dst_v)

    iota = lax.iota(jnp.int32, 16)
    bankoff = (iota % 8) * NPAD
    ones = jnp.ones((16,), _f32)
    m0 = iota < 8
    m1 = jnp.logical_not(m0)

    @pl.loop(0, NB)
    def _(b):
        for k in range(BE // 16):
            v = dst_v[b, pl.ds(k * 16, 16)]
            idx = v + bankoff
            plsc.addupdate_scatter(hist_v, [idx], ones, mask=m0)
            plsc.addupdate_scatter(hist_v, [idx], ones, mask=m1)

    @pl.loop(0, NPAD // 16)
    def _(g):
        acc = hist_v[pl.ds(g * 16, 16)]
        for bk in range(1, 8):
            acc = acc + hist_v[pl.ds(bk * NPAD + g * 16, 16)]
        red_v[0, pl.ds(g * 16, 16)] = acc

    pltpu.sync_copy(red_v, out_hbm.at[wid])


def _deg_call(dst3):
    import dataclasses
    cp = pltpu.CompilerParams()
    if "needs_layout_passes" in pltpu.CompilerParams.__dataclass_fields__:
        cp = dataclasses.replace(cp, needs_layout_passes=False)
    f = pl.kernel(
        _deg_body,
        out_type=jax.ShapeDtypeStruct((NW, 1, NPAD), _f32),
        mesh=_MESH,
        compiler_params=cp,
        scratch_types=[
            pltpu.VMEM((NB, BE), jnp.int32),
            pltpu.VMEM((8 * NPAD,), _f32),
            pltpu.VMEM((1, NPAD), _f32),
        ],
    )
    return f(dst3)


def _msg_body(hw_hbm, src_hbm, dst_hbm, out0_hbm, out1_hbm,
              src_a, dst_a, src_b, dst_b, r0, r1, r2, r3, acc_sh,
              g0, g1, g2, g3, s0, s1, s2, s3, semi):
    rows = (r0, r1, r2, r3)
    gs = (g0, g1, g2, g3)
    ss = (s0, s1, s2, s3)
    idxs = ((src_a, dst_a), (src_b, dst_b))
    cid = lax.axis_index("c")
    sid = lax.axis_index("s")
    wid = cid * NS + sid
    base = sid * RPW

    _zero_stripe(r0, acc_sh, base)
    plsc.subcore_barrier()

    def sref(b):
        return idxs[(b // CH) % 2][0].at[b % CH]

    def dref(b):
        return idxs[(b // CH) % 2][1].at[b % CH]

    pltpu.sync_copy(src_hbm.at[wid, pl.ds(0, CH)], src_a)
    pltpu.sync_copy(dst_hbm.at[wid, pl.ds(0, CH)], dst_a)
    pltpu.async_copy(hw_hbm.at[sref(0)], rows[0], gs[0])
    pltpu.async_copy(hw_hbm.at[sref(1)], rows[1], gs[1])

    for i in range(NB):
        j = i % 4
        if i % CH == 0 and i + CH < NB:
            c1 = i // CH + 1
            ns, nd = idxs[c1 % 2]
            pltpu.async_copy(src_hbm.at[wid, pl.ds(c1 * CH, CH)], ns, semi)
            pltpu.async_copy(dst_hbm.at[wid, pl.ds(c1 * CH, CH)], nd, semi)
        if i % CH == CH - 4 and i + CH < NB:
            c1 = i // CH + 1
            ns, nd = idxs[c1 % 2]
            pltpu.make_async_copy(src_hbm.at[wid, pl.ds(c1 * CH, CH)], ns, semi).wait()
            pltpu.make_async_copy(dst_hbm.at[wid, pl.ds(c1 * CH, CH)], nd, semi).wait()

        pltpu.make_async_copy(hw_hbm.at[sref(i)], rows[j], gs[j]).wait()
        pltpu.async_copy(rows[j], acc_sh.at[dref(i)], ss[j], add=True)

        bp = i + 2
        if bp < NB:
            jp = bp % 4
            if bp >= 4:
                pltpu.make_async_copy(rows[jp], acc_sh.at[dref(bp - 4)],
                                      ss[jp]).wait()
            pltpu.async_copy(hw_hbm.at[sref(bp)], rows[jp], gs[jp])

    for b in range(NB - 4, NB):
        j = b % 4
        pltpu.make_async_copy(rows[j], acc_sh.at[dref(b)], ss[j]).wait()

    plsc.subcore_barrier()

    @pl.when(cid == 0)
    def _():
        pltpu.sync_copy(acc_sh.at[pl.ds(base, RPW)],
                        out0_hbm.at[pl.ds(base, RPW)])

    @pl.when(cid == 1)
    def _():
        pltpu.sync_copy(acc_sh.at[pl.ds(base, RPW)],
                        out1_hbm.at[pl.ds(base, RPW)])


def _msg_call(hw, src3, dst3):
    f = pl.kernel(
        _msg_body,
        out_type=[jax.ShapeDtypeStruct((NPAD, D), _f32),
                  jax.ShapeDtypeStruct((NPAD, D), _f32)],
        mesh=_MESH,
        scratch_types=[
            pltpu.VMEM((CH, BE), jnp.int32),
            pltpu.VMEM((CH, BE), jnp.int32),
            pltpu.VMEM((CH, BE), jnp.int32),
            pltpu.VMEM((CH, BE), jnp.int32),
            pltpu.VMEM((BE, D), _f32),
            pltpu.VMEM((BE, D), _f32),
            pltpu.VMEM((BE, D), _f32),
            pltpu.VMEM((BE, D), _f32),
            pltpu.VMEM_SHARED((NPAD, D), _f32),
            pltpu.SemaphoreType.DMA,
            pltpu.SemaphoreType.DMA,
            pltpu.SemaphoreType.DMA,
            pltpu.SemaphoreType.DMA,
            pltpu.SemaphoreType.DMA,
            pltpu.SemaphoreType.DMA,
            pltpu.SemaphoreType.DMA,
            pltpu.SemaphoreType.DMA,
            pltpu.SemaphoreType.DMA,
        ],
    )
    return f(hw, src3, dst3)


def _dw_from_coeffs(cpad, ipad):
    r = ipad[:NSPEC_P]
    s = ipad[NSPEC_P:]
    j_a = lax.broadcasted_iota(jnp.int32, (D, NSPEC_P), 0)
    j_b = lax.broadcasted_iota(jnp.int32, (NSPEC_P, D), 1)
    scale = _f32(2.0 * np.pi / D)
    ang_a = ((j_a * r[None, :]) % D).astype(_f32) * scale
    ang_b = ((s[:, None] * j_b) % D).astype(_f32) * scale
    ca = jnp.cos(ang_a) * cpad[None, :]
    sa = jnp.sin(ang_a) * cpad[None, :]
    cb = jnp.cos(ang_b)
    sb = jnp.sin(ang_b)
    dw = (jnp.dot(ca, cb, preferred_element_type=_f32, precision=_HIGH)
          - jnp.dot(sa, sb, preferred_element_type=_f32, precision=_HIGH))
    return dw * _f32(ALPHA / (D * D))


def _fourier_body(x_ref, w1_ref, c1_ref, i1_ref, c2_ref, i2_ref,
                  hw1_ref, embf_ref, dw1_s, dw2_s):
    @pl.when(pl.program_id(0) == 0)
    def _():
        dw1_s[...] = _dw_from_coeffs(c1_ref[0], i1_ref[0])
        dw2_s[...] = _dw_from_coeffs(c2_ref[0], i2_ref[0])

    xb = x_ref[...]
    hw1_ref[...] = jnp.dot(xb, w1_ref[...],
                           preferred_element_type=_f32, precision=_HIGH)
    xf = jnp.maximum(
        jnp.dot(xb, dw1_s[...], preferred_element_type=_f32, precision=_HIGH),
        0.0)
    embf_ref[...] = jnp.dot(xf, dw2_s[...],
                            preferred_element_type=_f32, precision=_HIGH)


def _fourier_call(x, w1, cp1, ip1, cp2, ip2):
    full = lambda shape: pl.BlockSpec(shape, lambda i: (0, 0))
    return pl.pallas_call(
        _fourier_body,
        grid=(GRID,),
        in_specs=[
            pl.BlockSpec((BROW, D), lambda i: (i, 0)),
            full((D, D)),
            full((1, NSPEC_P)),
            full((1, 2 * NSPEC_P)),
            full((1, NSPEC_P)),
            full((1, 2 * NSPEC_P)),
        ],
        out_specs=[
            pl.BlockSpec((BROW, D), lambda i: (i, 0)),
            pl.BlockSpec((BROW, D), lambda i: (i, 0)),
        ],
        out_shape=[
            jax.ShapeDtypeStruct((N, D), _f32),
            jax.ShapeDtypeStruct((N, D), _f32),
        ],
        scratch_shapes=[
            pltpu.VMEM((D, D), _f32),
            pltpu.VMEM((D, D), _f32),
        ],
    )(x, w1, cp1, ip1, cp2, ip2)


def _prep_body(ht_ref, hw1_ref, dinv_ref, hw1s_ref):
    deg = jnp.sum(ht_ref[...], axis=1, keepdims=True) + 1.0
    dinv = lax.rsqrt(deg)
    dinv_b = jnp.broadcast_to(dinv, hw1_ref.shape)
    dinv_ref[...] = dinv_b
    hw1s_ref[...] = hw1_ref[...] * dinv_b


def _prep_call(ht, hw1):
    return pl.pallas_call(
        _prep_body,
        grid=(GRID,),
        in_specs=[
            pl.BlockSpec((BROW, NW), lambda i: (i, 0)),
            pl.BlockSpec((BROW, D), lambda i: (i, 0)),
        ],
        out_specs=[
            pl.BlockSpec((BROW, D), lambda i: (i, 0)),
            pl.BlockSpec((BROW, D), lambda i: (i, 0)),
        ],
        out_shape=[
            jax.ShapeDtypeStruct((N, D), _f32),
            jax.ShapeDtypeStruct((N, D), _f32),
        ],
    )(ht, hw1)


def _mid_body(a0_ref, a1_ref, hw1s_ref, dinv_ref, b1_ref, w2_ref, hw2s_ref):
    h1 = jnp.maximum(
        dinv_ref[...] * (a0_ref[...] + a1_ref[...] + hw1s_ref[...])
        + b1_ref[...], 0.0)
    hw2s_ref[...] = dinv_ref[...] * jnp.dot(
        h1, w2_ref[...], preferred_element_type=_f32, precision=_HIGH)


def _mid_call(a0, a1, hw1s, dinv, b1r, w2):
    row = pl.BlockSpec((BROW, D), lambda i: (i, 0))
    return pl.pallas_call(
        _mid_body,
        grid=(GRID,),
        in_specs=[row, row, row, row,
                  pl.BlockSpec((1, D), lambda i: (0, 0)),
                  pl.BlockSpec((D, D), lambda i: (0, 0))],
        out_specs=row,
        out_shape=jax.ShapeDtypeStruct((N, D), _f32),
    )(a0, a1, hw1s, dinv, b1r, w2)


def _final_body(a0_ref, a1_ref, hw2s_ref, dinv_ref, b2_ref, embf_ref,
                base_ref, total_ref):
    base = (dinv_ref[...] * (a0_ref[...] + a1_ref[...] + hw2s_ref[...])
            + b2_ref[...])
    base_ref[...] = base
    total_ref[...] = base + embf_ref[...]


def _final_call(a0, a1, hw2s, dinv, b2r, embf):
    row = pl.BlockSpec((BROW, D), lambda i: (i, 0))
    return pl.pallas_call(
        _final_body,
        grid=(GRID,),
        in_specs=[row, row, row, row,
                  pl.BlockSpec((1, D), lambda i: (0, 0)),
                  row],
        out_specs=[row, row],
        out_shape=[
            jax.ShapeDtypeStruct((N, D), _f32),
            jax.ShapeDtypeStruct((N, D), _f32),
        ],
    )(a0, a1, hw2s, dinv, b2r, embf)


def kernel(x, edge_index, W1, b1, W2, b2, c1, c2, idx1, idx2):
    src = edge_index[0]
    dst = edge_index[1]
    npad = EP - E
    pad_ids = jnp.arange(npad, dtype=jnp.int32)
    psrc = jnp.concatenate([src, pad_ids % 997])
    pdst = jnp.concatenate([dst, N + (pad_ids % 112)])
    src3 = psrc.reshape(NW, NB, BE)
    dst3 = pdst.reshape(NW, NB, BE)

    def pack(c, idx):
        cp = jnp.zeros((1, NSPEC_P), _f32).at[0, :NSPEC].set(c)
        ip = jnp.zeros((1, 2 * NSPEC_P), jnp.int32)
        ip = ip.at[0, :NSPEC].set(idx[0]).at[0, NSPEC_P:NSPEC_P + NSPEC].set(idx[1])
        return cp, ip

    cp1, ip1 = pack(c1, idx1)
    cp2, ip2 = pack(c2, idx2)

    h3 = _deg_call(dst3)
    hw1, embf = _fourier_call(x, W1, cp1, ip1, cp2, ip2)

    ht = h3.reshape(NW, NPAD).T
    dinv, hw1s = _prep_call(ht, hw1)

    a10, a11 = _msg_call(hw1s, src3, dst3)
    hw2s = _mid_call(a10, a11, hw1s, dinv, b1.reshape(1, D), W2)

    a20, a21 = _msg_call(hw2s, src3, dst3)
    emb_base, emb_total = _final_call(a20, a21, hw2s, dinv,
                                      b2.reshape(1, D), embf)
    return (emb_total, emb_base, embf)
